# baseline (speedup 1.0000x reference)
import functools

import jax
import jax.numpy as jnp
from jax.experimental import pallas as pl
from jax.experimental.pallas import tpu as pltpu


def _make_conv_body(H, W, K, Cin, C1, C2, NB):
    pad = K // 2
    Ho, Wo = H // 2, W // 2
    Ho2, Wo2 = Ho // 2, Wo // 2
    SL = (K + 1) * Cin
    KC1 = (K + 1) * C1
    I1 = H // 2 + pad
    I2 = Ho // 2 + pad

    def body(xs_ref, w1_ref, b1_ref, w2_ref, b2_ref, out_ref, s_ref):
        acc = jnp.dot(xs_ref[...].reshape(NB * Ho * Wo, (K + 1) * SL),
                      w1_ref[...], preferred_element_type=jnp.float32)
        m1 = jnp.maximum(acc[:, 0:2 * C1], acc[:, 2 * C1:4 * C1])
        y1 = jnp.maximum(jnp.maximum(m1[:, 0:C1], m1[:, C1:2 * C1])
                         + b1_ref[...], 0.0)

        zrow = jnp.zeros((NB, 2, Wo2, KC1), jnp.float32)
        s_ref[:, :, 0, :, :] = zrow
        s_ref[:, :, I2 - 1, :, :] = zrow
        zcol = jnp.zeros((NB, 2, Ho2, 1, C1), jnp.float32)
        for kw6 in range(K + 1):
            sh = kw6 // 2 - 1
            if sh == -1:
                s_ref[:, :, 1:1 + Ho2, 0:1, kw6 * C1:(kw6 + 1) * C1] = zcol
            elif sh == 1:
                s_ref[:, :, 1:1 + Ho2, Wo2 - 1:Wo2,
                      kw6 * C1:(kw6 + 1) * C1] = zcol
        for par in range(2):
            t = (y1.reshape(NB, Ho2, 2, Wo, C1)[:, :, par]
                 .reshape(NB, Ho2, 2, Wo2, C1))
            for kw6 in range(K + 1):
                sh = kw6 // 2 - 1
                lo, hi = max(0, -sh), min(Wo2, Wo2 - sh)
                s_ref[:, par, 1:1 + Ho2, lo:hi,
                      kw6 * C1:(kw6 + 1) * C1] = (
                          t[:, :, kw6 % 2, lo + sh:hi + sh, :])

        zz = []
        for e in range(2):
            acc2 = None
            for kh in range(K):
                u = e + kh
                lhs = s_ref[:, u % 2, u // 2:u // 2 + Ho2, :, :]
                d = jnp.dot(lhs.reshape(NB * Ho2 * Wo2, KC1),
                            w2_ref[kh * KC1:(kh + 1) * KC1, :],
                            preferred_element_type=jnp.float32)
                acc2 = d if acc2 is None else acc2 + d
            zz.append(acc2)
        m = jnp.maximum(zz[0], zz[1])
        y2 = jnp.maximum(jnp.maximum(m[:, 0:C2], m[:, C2:2 * C2])
                         + b2_ref[...], 0.0)
        out_ref[...] = y2.reshape(NB, Ho2, Wo2, C2).astype(jnp.bfloat16)

    return body


def _fc_body(x_ref, w_ref, b_ref, out_ref, wb_ref):
    wb_ref[...] = w_ref[...].astype(jnp.bfloat16)
    acc = jnp.dot(x_ref[...], wb_ref[...],
                  preferred_element_type=jnp.float32)
    out_ref[...] = jnp.maximum(acc + b_ref[...], 0.0)


@functools.partial(jax.jit, static_argnames=("K", "fc_out"))
def _forward(x_nchw, w1_mat, b1_r, w2_mat, b2_r, wfc_mat, bfc_r, *,
             K=5, fc_out=500):
    B, Cin, H, W = x_nchw.shape
    pad = K // 2
    C1 = w1_mat.shape[1]
    C2 = w2_mat.shape[1]
    Ho2, Wo2 = H // 4, W // 4
    fc_in = Ho2 * Wo2 * C2
    fc_out_pad = wfc_mat.shape[1]
    SL = (K + 1) * Cin
    KC1 = (K + 1) * C1
    I1 = H // 2 + pad

    Ho = H // 2
    xt = jnp.transpose(x_nchw, (0, 2, 3, 1))
    xp = jnp.pad(xt, ((0, 0), (pad, pad), (pad, pad), (0, 0)))
    xpr = xp.reshape(B, H + 2 * pad, (W + 2 * pad) // 4, 4, Cin)
    cols = []
    for wpar in range(2):
        pieces = []
        for kw6 in range(K + 1):
            c0 = kw6 + 2 * wpar
            pieces.append(xpr[:, :, c0 // 4:c0 // 4 + Wo2, c0 % 4, :])
        base = jnp.concatenate(pieces, axis=-1)
        rows = [base[:, u:u + H:2] for u in range(K + 1)]
        cols.append(jnp.concatenate(rows, axis=-1))
    xs = jnp.stack(cols, axis=2).reshape(
        B, Ho, 2 * Wo2, (K + 1) * SL).astype(jnp.bfloat16)

    w1r = w1_mat.reshape(K, K, Cin, C1)
    blocks = []
    for a in range(2):
        for b in range(2):
            wp = jnp.pad(w1r, ((a, 1 - a), (b, 1 - b), (0, 0), (0, 0)))
            blocks.append(wp.reshape((K + 1) * SL, C1))
    w1x = jnp.concatenate(blocks, axis=1).astype(jnp.bfloat16)

    w2r = w2_mat.reshape(K, K, C1, C2)
    f0 = jnp.pad(w2r, ((0, 0), (0, 1), (0, 0), (0, 0)))
    f1 = jnp.pad(w2r, ((0, 0), (1, 0), (0, 0), (0, 0)))
    w2x = jnp.concatenate([f0.reshape(K, KC1, C2),
                           f1.reshape(K, KC1, C2)], axis=2)
    w2x = w2x.reshape(K * KC1, 2 * C2)

    NB = 8 if B % 8 == 0 else 1
    conv_body = _make_conv_body(H, W, K, Cin, C1, C2, NB)
    y2 = pl.pallas_call(
        conv_body,
        grid=(B // NB,),
        in_specs=[
            pl.BlockSpec((NB, H // 2, W // 2, (K + 1) * SL),
                         lambda b: (b, 0, 0, 0)),
            pl.BlockSpec(((K + 1) * SL, 4 * C1), lambda b: (0, 0)),
            pl.BlockSpec((1, C1), lambda b: (0, 0)),
            pl.BlockSpec((K * KC1, 2 * C2), lambda b: (0, 0)),
            pl.BlockSpec((1, C2), lambda b: (0, 0)),
        ],
        out_specs=pl.BlockSpec((NB, Ho2, Wo2, C2), lambda b: (b, 0, 0, 0)),
        out_shape=jax.ShapeDtypeStruct((B, Ho2, Wo2, C2), jnp.bfloat16),
        scratch_shapes=[
            pltpu.VMEM((NB, 2, H // 4 + pad, Wo2, KC1), jnp.float32),
        ],
        compiler_params=pltpu.CompilerParams(
            dimension_semantics=("parallel",)),
    )(xs, w1x, b1_r, w2x, b2_r)

    flat = y2.reshape(B, fc_in)

    n_blk = 2 if (fc_out_pad % 256 == 0) else 1
    blk = fc_out_pad // n_blk
    z = pl.pallas_call(
        _fc_body,
        grid=(n_blk,),
        in_specs=[
            pl.BlockSpec((B, fc_in), lambda j: (0, 0)),
            pl.BlockSpec((fc_in, blk), lambda j: (0, j)),
            pl.BlockSpec((1, blk), lambda j: (0, j)),
        ],
        out_specs=pl.BlockSpec((B, blk), lambda j: (0, j)),
        out_shape=jax.ShapeDtypeStruct((B, fc_out_pad), jnp.float32),
        scratch_shapes=[pltpu.VMEM((fc_in, blk), jnp.bfloat16)],
        compiler_params=pltpu.CompilerParams(
            dimension_semantics=("parallel",)),
    )(flat, wfc_mat, bfc_r)
    return z[:, :fc_out]


def kernel(x, w1_mat, b1_r, w2_mat, b2_r, wfc_mat, bfc_r):
    return _forward(x, w1_mat, b1_r, w2_mat, b2_r, wfc_mat, bfc_r,
                    K=5, fc_out=500)


# speedup vs baseline: 1.0214x; 1.0214x over previous
import functools

import jax
import jax.numpy as jnp
from jax.experimental import pallas as pl
from jax.experimental.pallas import tpu as pltpu


def _make_conv_body(H, W, K, Cin, C1, C2, NB):
    pad = K // 2
    Ho, Wo = H // 2, W // 2
    Ho2, Wo2 = Ho // 2, Wo // 2
    SL = (K + 1) * Cin
    KC1 = (K + 1) * C1
    I1 = H // 2 + pad
    I2 = Ho // 2 + pad

    def body(xs_ref, w1_ref, b1_ref, w2_ref, b2_ref, out_ref, s_ref):
        acc = jnp.dot(xs_ref[...].reshape(NB * Ho * Wo, (K + 1) * SL),
                      w1_ref[...], preferred_element_type=jnp.float32)
        y1 = jnp.maximum(
            jnp.maximum(jnp.maximum(acc[:, 0:C1], acc[:, C1:2 * C1]),
                        jnp.maximum(acc[:, 2 * C1:3 * C1],
                                    acc[:, 3 * C1:4 * C1]))
            + b1_ref[...], 0.0)

        s_ref[...] = jnp.zeros((NB, 2, I2, Wo2, KC1), jnp.float32)
        for par in range(2):
            t = (y1.reshape(NB, Ho2, 2, Wo, C1)[:, :, par]
                 .reshape(NB, Ho2, 2, Wo2, C1))
            for kw6 in range(K + 1):
                sh = kw6 // 2 - 1
                lo, hi = max(0, -sh), min(Wo2, Wo2 - sh)
                s_ref[:, par, 1:1 + Ho2, lo:hi,
                      kw6 * C1:(kw6 + 1) * C1] = (
                          t[:, :, kw6 % 2, lo + sh:hi + sh, :])

        zz = []
        for e in range(2):
            acc2 = None
            for kh in range(K):
                u = e + kh
                lhs = s_ref[:, u % 2, u // 2:u // 2 + Ho2, :, :]
                d = jnp.dot(lhs.reshape(NB * Ho2 * Wo2, KC1),
                            w2_ref[kh * KC1:(kh + 1) * KC1, :],
                            preferred_element_type=jnp.float32)
                acc2 = d if acc2 is None else acc2 + d
            zz.append(acc2)
        m = jnp.maximum(zz[0], zz[1])
        y2 = jnp.maximum(jnp.maximum(m[:, 0:C2], m[:, C2:2 * C2])
                         + b2_ref[...], 0.0)
        out_ref[...] = y2.reshape(NB, Ho2, Wo2, C2).astype(jnp.bfloat16)

    return body


def _fc_body(x_ref, w_ref, b_ref, out_ref, wb_ref):
    wb_ref[...] = w_ref[...].astype(jnp.bfloat16)
    acc = jnp.dot(x_ref[...], wb_ref[...],
                  preferred_element_type=jnp.float32)
    out_ref[...] = jnp.maximum(acc + b_ref[...], 0.0)


@functools.partial(jax.jit, static_argnames=("K", "fc_out"))
def _forward(x_nchw, w1_mat, b1_r, w2_mat, b2_r, wfc_mat, bfc_r, *,
             K=5, fc_out=500):
    B, Cin, H, W = x_nchw.shape
    pad = K // 2
    C1 = w1_mat.shape[1]
    C2 = w2_mat.shape[1]
    Ho2, Wo2 = H // 4, W // 4
    fc_in = Ho2 * Wo2 * C2
    fc_out_pad = wfc_mat.shape[1]
    SL = (K + 1) * Cin
    KC1 = (K + 1) * C1
    I1 = H // 2 + pad

    Ho = H // 2
    xt = jnp.transpose(x_nchw, (0, 2, 3, 1))
    xp = jnp.pad(xt, ((0, 0), (pad, pad), (pad, pad), (0, 0)))
    xpr = xp.reshape(B, H + 2 * pad, (W + 2 * pad) // 4, 4, Cin)
    cols = []
    for wpar in range(2):
        pieces = []
        for kw6 in range(K + 1):
            c0 = kw6 + 2 * wpar
            pieces.append(xpr[:, :, c0 // 4:c0 // 4 + Wo2, c0 % 4, :])
        base = jnp.concatenate(pieces, axis=-1)
        rows = [base[:, u:u + H:2] for u in range(K + 1)]
        cols.append(jnp.concatenate(rows, axis=-1))
    xs = jnp.stack(cols, axis=2).reshape(
        B, Ho, 2 * Wo2, (K + 1) * SL).astype(jnp.bfloat16)

    w1r = w1_mat.reshape(K, K, Cin, C1)
    blocks = []
    for a in range(2):
        for b in range(2):
            wp = jnp.pad(w1r, ((a, 1 - a), (b, 1 - b), (0, 0), (0, 0)))
            blocks.append(wp.reshape((K + 1) * SL, C1))
    w1x = jnp.concatenate(blocks, axis=1).astype(jnp.bfloat16)

    w2r = w2_mat.reshape(K, K, C1, C2)
    f0 = jnp.pad(w2r, ((0, 0), (0, 1), (0, 0), (0, 0)))
    f1 = jnp.pad(w2r, ((0, 0), (1, 0), (0, 0), (0, 0)))
    w2x = jnp.concatenate([f0.reshape(K, KC1, C2),
                           f1.reshape(K, KC1, C2)], axis=2)
    w2x = w2x.reshape(K * KC1, 2 * C2)

    NB = 16 if B % 16 == 0 else 1
    conv_body = _make_conv_body(H, W, K, Cin, C1, C2, NB)
    y2 = pl.pallas_call(
        conv_body,
        grid=(B // NB,),
        in_specs=[
            pl.BlockSpec((NB, H // 2, W // 2, (K + 1) * SL),
                         lambda b: (b, 0, 0, 0)),
            pl.BlockSpec(((K + 1) * SL, 4 * C1), lambda b: (0, 0)),
            pl.BlockSpec((1, C1), lambda b: (0, 0)),
            pl.BlockSpec((K * KC1, 2 * C2), lambda b: (0, 0)),
            pl.BlockSpec((1, C2), lambda b: (0, 0)),
        ],
        out_specs=pl.BlockSpec((NB, Ho2, Wo2, C2), lambda b: (b, 0, 0, 0)),
        out_shape=jax.ShapeDtypeStruct((B, Ho2, Wo2, C2), jnp.bfloat16),
        scratch_shapes=[
            pltpu.VMEM((NB, 2, H // 4 + pad, Wo2, KC1), jnp.float32),
        ],
        compiler_params=pltpu.CompilerParams(
            dimension_semantics=("parallel",)),
    )(xs, w1x, b1_r, w2x, b2_r)

    flat = y2.reshape(B, fc_in)

    n_blk = 2 if (fc_out_pad % 256 == 0) else 1
    blk = fc_out_pad // n_blk
    z = pl.pallas_call(
        _fc_body,
        grid=(n_blk,),
        in_specs=[
            pl.BlockSpec((B, fc_in), lambda j: (0, 0)),
            pl.BlockSpec((fc_in, blk), lambda j: (0, j)),
            pl.BlockSpec((1, blk), lambda j: (0, j)),
        ],
        out_specs=pl.BlockSpec((B, blk), lambda j: (0, j)),
        out_shape=jax.ShapeDtypeStruct((B, fc_out_pad), jnp.float32),
        scratch_shapes=[pltpu.VMEM((fc_in, blk), jnp.bfloat16)],
        compiler_params=pltpu.CompilerParams(
            dimension_semantics=("parallel",)),
    )(flat, wfc_mat, bfc_r)
    return z[:, :fc_out]


def kernel(x, w1_mat, b1_r, w2_mat, b2_r, wfc_mat, bfc_r):
    return _forward(x, w1_mat, b1_r, w2_mat, b2_r, wfc_mat, bfc_r,
                    K=5, fc_out=500)
